# Initial kernel scaffold; baseline (speedup 1.0000x reference)
import jax
import jax.numpy as jnp
from jax.experimental import pallas as pl


def kernel(p, x, o, W2, g2, b2, Ws1, gs1, bs1, Ws2, bs2):
    raise NotImplementedError("write your pallas kernel here")


# speedup vs baseline: 10.3673x; 10.3673x over previous
import functools

import jax
import jax.numpy as jnp
from jax import lax
from jax.experimental import pallas as pl
from jax.experimental.pallas import tpu as pltpu
from jax.experimental.pallas import tpu_sc as plsc

N = 8192
C_IN = 128
C_OUT = 256
K = 16
M = N // 4
EPS = 1e-5
BIGI = 2**30


_FR, _FC = 64, 128


def _fps_body(px_ref, py_ref, pz_ref, npx_ref, npy_ref, npz_ref):
    rows = lax.broadcasted_iota(jnp.int32, (_FR, _FC), 0)
    cols = lax.broadcasted_iota(jnp.int32, (_FR, _FC), 1)
    lin = rows * _FC + cols
    px = px_ref[...]
    py = py_ref[...]
    pz = pz_ref[...]
    qx0 = jnp.sum(jnp.where(lin == 0, px, 0.0))
    qy0 = jnp.sum(jnp.where(lin == 0, py, 0.0))
    qz0 = jnp.sum(jnp.where(lin == 0, pz, 0.0))
    npx_ref[0] = qx0
    npy_ref[0] = qy0
    npz_ref[0] = qz0

    def step(i, carry):
        dists, qx, qy, qz = carry
        dx = px - qx
        dy = py - qy
        dz = pz - qz
        d = dx * dx + dy * dy + dz * dz
        dists = jnp.minimum(dists, d)
        mx = jnp.max(dists)
        nxt = jnp.min(jnp.where(dists == mx, lin, BIGI))
        nx = jnp.sum(jnp.where(lin == nxt, px, 0.0))
        ny = jnp.sum(jnp.where(lin == nxt, py, 0.0))
        nz = jnp.sum(jnp.where(lin == nxt, pz, 0.0))
        npx_ref[i] = nx
        npy_ref[i] = ny
        npz_ref[i] = nz
        return (dists, nx, ny, nz)

    init = (jnp.full((_FR, _FC), 1e10, jnp.float32), qx0, qy0, qz0)
    lax.fori_loop(1, M, step, init)


def _run_fps(p):
    px = p[:, 0].reshape(_FR, _FC)
    py = p[:, 1].reshape(_FR, _FC)
    pz = p[:, 2].reshape(_FR, _FC)
    return pl.pallas_call(
        _fps_body,
        out_shape=[jax.ShapeDtypeStruct((M,), jnp.float32)] * 3,
        out_specs=[pl.BlockSpec(memory_space=pltpu.SMEM)] * 3,
    )(px, py, pz)


_KT = 128
_KC = 1024
_NKC = N // _KC


def _knn_body(np_ref, pT_ref, knn_ref, d2_ref):
    cx = np_ref[:, 0:1]
    cy = np_ref[:, 1:2]
    cz = np_ref[:, 2:3]
    for c in range(_NKC):
        s = c * _KC
        dx = cx - pT_ref[0:1, s:s + _KC]
        dy = cy - pT_ref[1:2, s:s + _KC]
        dz = cz - pT_ref[2:3, s:s + _KC]
        d2_ref[:, s:s + _KC] = dx * dx + dy * dy + dz * dz
    inf = jnp.float32(jnp.inf)
    im = jnp.full((_KT, 1), -1, jnp.int32)
    for k in range(K):
        mn = jnp.full((_KT, 1), inf, jnp.float32)
        for c in range(_NKC):
            s = c * _KC
            ci = lax.broadcasted_iota(jnp.int32, (_KT, _KC), 1) + s
            blk = d2_ref[:, s:s + _KC]
            if k > 0:
                blk = jnp.where(ci == im, inf, blk)
                d2_ref[:, s:s + _KC] = blk
            mn = jnp.minimum(mn, jnp.min(blk, axis=1, keepdims=True))
        im = jnp.full((_KT, 1), BIGI, jnp.int32)
        for c in range(_NKC):
            s = c * _KC
            ci = lax.broadcasted_iota(jnp.int32, (_KT, _KC), 1) + s
            blk = d2_ref[:, s:s + _KC]
            cand = jnp.min(jnp.where(blk == mn, ci, BIGI), axis=1, keepdims=True)
            im = jnp.minimum(im, cand)
        knn_ref[:, k:k + 1] = im


def _run_knn(n_p, pT):
    return pl.pallas_call(
        _knn_body,
        grid=(M // _KT,),
        in_specs=[
            pl.BlockSpec((_KT, 3), lambda i: (i, 0)),
            pl.BlockSpec((3, N), lambda i: (0, 0)),
        ],
        out_specs=pl.BlockSpec((_KT, K), lambda i: (i, 0)),
        out_shape=jax.ShapeDtypeStruct((M, K), jnp.int32),
        scratch_shapes=[pltpu.VMEM((_KT, N), jnp.float32)],
    )(n_p, pT)


_MMB = 512
_NMM = N // _MMB


def _mm_body(x_ref, p_ref, w2_ref, wsp_ref, a_ref, h2_ref, u_ref, sums_ref,
             acc_ref):
    i = pl.program_id(0)
    xb = x_ref[...]
    h2 = jnp.dot(xb, w2_ref[...], preferred_element_type=jnp.float32)
    u = (jnp.dot(xb, wsp_ref[...], preferred_element_type=jnp.float32)
         + jnp.dot(p_ref[...], a_ref[...], preferred_element_type=jnp.float32))
    h2_ref[...] = h2
    u_ref[...] = u
    s1 = jnp.sum(h2, axis=0, keepdims=True)
    s2 = jnp.sum(h2 * h2, axis=0, keepdims=True)

    @pl.when(i == 0)
    def _():
        acc_ref[0:1, :] = s1
        acc_ref[1:2, :] = s2

    @pl.when(i > 0)
    def _():
        acc_ref[0:1, :] = acc_ref[0:1, :] + s1
        acc_ref[1:2, :] = acc_ref[1:2, :] + s2

    @pl.when(i == _NMM - 1)
    def _():
        sums_ref[...] = acc_ref[...]


def _run_mm(x, p, W2, Ws1p, A):
    return pl.pallas_call(
        _mm_body,
        grid=(_NMM,),
        in_specs=[
            pl.BlockSpec((_MMB, C_IN), lambda i: (i, 0)),
            pl.BlockSpec((_MMB, 3), lambda i: (i, 0)),
            pl.BlockSpec((C_IN, C_OUT), lambda i: (0, 0)),
            pl.BlockSpec((C_IN, C_IN), lambda i: (0, 0)),
            pl.BlockSpec((3, C_IN), lambda i: (0, 0)),
        ],
        out_specs=[
            pl.BlockSpec((_MMB, C_OUT), lambda i: (i, 0)),
            pl.BlockSpec((_MMB, C_IN), lambda i: (i, 0)),
            pl.BlockSpec((2, C_OUT), lambda i: (0, 0)),
        ],
        out_shape=[
            jax.ShapeDtypeStruct((N, C_OUT), jnp.float32),
            jax.ShapeDtypeStruct((N, C_IN), jnp.float32),
            jax.ShapeDtypeStruct((2, C_OUT), jnp.float32),
        ],
        scratch_shapes=[pltpu.VMEM((2, C_OUT), jnp.float32)],
    )(x, p, W2, Ws1p, A)


def _bnrelu_body(h_ref, sums_ref, g_ref, b_ref, y_ref, *, n_rows):
    mu = sums_ref[0:1, :] / n_rows
    var = sums_ref[1:2, :] / n_rows - mu * mu
    y = g_ref[...] * (h_ref[...] - mu) / jnp.sqrt(var + EPS) + b_ref[...]
    y_ref[...] = jnp.maximum(y, 0.0)


def _run_bnrelu(h, sums, g, b, n_rows, blk):
    rows, cols = h.shape
    return pl.pallas_call(
        functools.partial(_bnrelu_body, n_rows=n_rows),
        grid=(rows // blk,),
        in_specs=[
            pl.BlockSpec((blk, cols), lambda i: (i, 0)),
            pl.BlockSpec((2, cols), lambda i: (0, 0)),
            pl.BlockSpec((1, cols), lambda i: (0, 0)),
            pl.BlockSpec((1, cols), lambda i: (0, 0)),
        ],
        out_specs=pl.BlockSpec((blk, cols), lambda i: (i, 0)),
        out_shape=jax.ShapeDtypeStruct((rows, cols), jnp.float32),
    )(h, sums, g, b)


_NC, _NS = 2, 16
_NW = _NC * _NS
_RW = (M * K) // _NW
_GC = 256


def _sc_gather_body(v_hbm, knn_hbm, vg_hbm, idx0, idx1, idx2, idx3,
                    buf0, buf1, sem0, sem1):
    wid = lax.axis_index("s") * _NC + lax.axis_index("c")
    base = wid * _RW
    idxs = (idx0, idx1, idx2, idx3)
    for c in range(_RW // _GC):
        pltpu.sync_copy(knn_hbm.at[pl.ds(base + c * _GC, _GC)], idxs[c])
    bufs = (buf0, buf1)
    sems = (sem0, sem1)
    cps = [None, None]
    cps[0] = pltpu.async_copy(v_hbm.at[idx0], buf0, sem0)
    for c in range(_RW // _GC):
        if c + 1 < _RW // _GC:
            cps[(c + 1) % 2] = pltpu.async_copy(
                v_hbm.at[idxs[c + 1]], bufs[(c + 1) % 2], sems[(c + 1) % 2])
        cps[c % 2].wait()
        pltpu.sync_copy(bufs[c % 2], vg_hbm.at[pl.ds(base + c * _GC, _GC)])


def _run_sc_gather(v, knn):
    mesh = plsc.VectorSubcoreMesh(core_axis_name="c", subcore_axis_name="s")
    f = pl.kernel(
        _sc_gather_body,
        out_type=jax.ShapeDtypeStruct((M * K, C_IN), jnp.float32),
        mesh=mesh,
        scratch_types=[
            pltpu.VMEM((_GC,), jnp.int32),
            pltpu.VMEM((_GC,), jnp.int32),
            pltpu.VMEM((_GC,), jnp.int32),
            pltpu.VMEM((_GC,), jnp.int32),
            pltpu.VMEM((_GC, C_IN), jnp.float32),
            pltpu.VMEM((_GC, C_IN), jnp.float32),
            pltpu.SemaphoreType.DMA,
            pltpu.SemaphoreType.DMA,
        ],
    )
    return f(v, knn.reshape(M * K))


_PB = 1024
_NPB = (M * K) // _PB


def _pre_body(vg_ref, np_ref, a_ref, pre_ref, sums_ref, acc_ref):
    i = pl.program_id(0)
    w = jnp.dot(np_ref[...], a_ref[...], preferred_element_type=jnp.float32)
    r16 = lax.broadcasted_iota(jnp.int32, (_PB, _PB // K), 0) // K
    c16 = lax.broadcasted_iota(jnp.int32, (_PB, _PB // K), 1)
    e = jnp.where(r16 == c16, 1.0, 0.0)
    w_exp = jnp.dot(e, w, preferred_element_type=jnp.float32)
    pre = vg_ref[...] - w_exp
    pre_ref[...] = pre
    s1 = jnp.sum(pre, axis=0, keepdims=True)
    s2 = jnp.sum(pre * pre, axis=0, keepdims=True)

    @pl.when(i == 0)
    def _():
        acc_ref[0:1, :] = s1
        acc_ref[1:2, :] = s2

    @pl.when(i > 0)
    def _():
        acc_ref[0:1, :] = acc_ref[0:1, :] + s1
        acc_ref[1:2, :] = acc_ref[1:2, :] + s2

    @pl.when(i == _NPB - 1)
    def _():
        sums_ref[...] = acc_ref[...]


def _run_pre(vg, n_p, A):
    return pl.pallas_call(
        _pre_body,
        grid=(_NPB,),
        in_specs=[
            pl.BlockSpec((_PB, C_IN), lambda i: (i, 0)),
            pl.BlockSpec((_PB // K, 3), lambda i: (i, 0)),
            pl.BlockSpec((3, C_IN), lambda i: (0, 0)),
        ],
        out_specs=[
            pl.BlockSpec((_PB, C_IN), lambda i: (i, 0)),
            pl.BlockSpec((2, C_IN), lambda i: (0, 0)),
        ],
        out_shape=[
            jax.ShapeDtypeStruct((M * K, C_IN), jnp.float32),
            jax.ShapeDtypeStruct((2, C_IN), jnp.float32),
        ],
        scratch_shapes=[pltpu.VMEM((2, C_IN), jnp.float32)],
    )(vg, n_p, A)


_HB = 2048


def _hs_body(pre_ref, sums_ref, g_ref, b_ref, w_ref, bs_ref, s_ref):
    n_rows = jnp.float32(M * K)
    mu = sums_ref[0:1, :] / n_rows
    var = sums_ref[1:2, :] / n_rows - mu * mu
    h = g_ref[...] * (pre_ref[...] - mu) / jnp.sqrt(var + EPS) + b_ref[...]
    h = jnp.maximum(h, 0.0)
    s_ref[...] = jnp.sum(h * w_ref[...], axis=1, keepdims=True) + bs_ref[0, 0]


def _run_hs(pre, sums, gs1, bs1, Ws2, bs2):
    return pl.pallas_call(
        _hs_body,
        grid=((M * K) // _HB,),
        in_specs=[
            pl.BlockSpec((_HB, C_IN), lambda i: (i, 0)),
            pl.BlockSpec((2, C_IN), lambda i: (0, 0)),
            pl.BlockSpec((1, C_IN), lambda i: (0, 0)),
            pl.BlockSpec((1, C_IN), lambda i: (0, 0)),
            pl.BlockSpec((1, C_IN), lambda i: (0, 0)),
            pl.BlockSpec((1, 1), lambda i: (0, 0), memory_space=pltpu.SMEM),
        ],
        out_specs=pl.BlockSpec((_HB, 1), lambda i: (i, 0)),
        out_shape=jax.ShapeDtypeStruct((M * K, 1), jnp.float32),
    )(pre, sums, gs1.reshape(1, C_IN), bs1.reshape(1, C_IN),
      Ws2.reshape(1, C_IN), bs2.reshape(1, 1))


def _softmax_body(s_ref, p_ref):
    s = s_ref[...]
    mx = jnp.max(s, axis=1, keepdims=True)
    e = jnp.exp(s - mx)
    p_ref[...] = e / jnp.sum(e, axis=1, keepdims=True)


def _run_softmax(s):
    return pl.pallas_call(
        _softmax_body,
        out_shape=jax.ShapeDtypeStruct((M, K), jnp.float32),
    )(s.reshape(M, K))


_MW = M // _NW
_GM = 4


def _sc_wsum_body(y_hbm, knn_hbm, prob_hbm, out_hbm, idxv, probv, buf0, buf1,
                  outb, sem0, sem1):
    wid = lax.axis_index("s") * _NC + lax.axis_index("c")
    base = wid * _MW * K
    pltpu.sync_copy(knn_hbm.at[pl.ds(base, _MW * K)], idxv)
    pltpu.sync_copy(prob_hbm.at[pl.ds(base, _MW * K)], probv)
    bufs = (buf0, buf1)
    sems = (sem0, sem1)
    dnums = lax.GatherDimensionNumbers(
        offset_dims=(), collapsed_slice_dims=(0,), start_index_map=(0,))

    def one_m(ml, buf):
        pm = probv[pl.ds(ml * K, K)]
        accs = [jnp.zeros((16,), jnp.float32) for _ in range(C_OUT // 16)]
        for k in range(K):
            ik = jnp.zeros((K, 1), jnp.int32) + k
            pk = lax.gather(pm, ik, dnums, (1,),
                            mode=lax.GatherScatterMode.PROMISE_IN_BOUNDS)
            for r in range(C_OUT // 16):
                accs[r] = accs[r] + pk * buf[k, pl.ds(r * 16, 16)]
        for r in range(C_OUT // 16):
            outb[ml, pl.ds(r * 16, 16)] = accs[r]

    def g_body(g, _):
        cps = []
        for j in range(2):
            iv = idxv[pl.ds((g * 2 + j) * K, K)]
            cps.append(pltpu.async_copy(y_hbm.at[iv], bufs[j], sems[j]))
        for j in range(2):
            cps[j].wait()
            one_m(g * 2 + j, bufs[j])
        return 0

    lax.fori_loop(0, _MW // 2, g_body, 0)
    pltpu.sync_copy(outb, out_hbm.at[pl.ds(wid * _MW, _MW)])


def _run_sc_wsum(y, knn, prob):
    mesh = plsc.VectorSubcoreMesh(core_axis_name="c", subcore_axis_name="s")
    f = pl.kernel(
        _sc_wsum_body,
        out_type=jax.ShapeDtypeStruct((M, C_OUT), jnp.float32),
        mesh=mesh,
        scratch_types=[
            pltpu.VMEM((_MW * K,), jnp.int32),
            pltpu.VMEM((_MW * K,), jnp.float32),
            pltpu.VMEM((K, C_OUT), jnp.float32),
            pltpu.VMEM((K, C_OUT), jnp.float32),
            pltpu.VMEM((_MW, C_OUT), jnp.float32),
            pltpu.SemaphoreType.DMA,
            pltpu.SemaphoreType.DMA,
        ],
    )
    return f(y, knn.reshape(M * K), prob.reshape(M * K))


def kernel(p, x, o, W2, g2, b2, Ws1, gs1, bs1, Ws2, bs2):
    npx, npy, npz = _run_fps(p)
    n_p = jnp.stack([npx, npy, npz], axis=1)
    knn = _run_knn(n_p, p.T)
    h2, v, sums2 = _run_mm(x, p, W2, Ws1[3:], Ws1[:3])
    y = _run_bnrelu(h2, sums2, g2.reshape(1, C_OUT), b2.reshape(1, C_OUT),
                    float(N), 512)
    vg = _run_sc_gather(v, knn)
    pre, sums1 = _run_pre(vg, n_p, Ws1[:3])
    s = _run_hs(pre, sums1, gs1, bs1, Ws2, bs2)
    prob = _run_softmax(s)
    y_out = _run_sc_wsum(y, knn, prob)
    n_o = jnp.array([M], dtype=jnp.int32)
    return (n_p, y_out, n_o)


# speedup vs baseline: 12.5146x; 1.2071x over previous
import functools

import jax
import jax.numpy as jnp
from jax import lax
from jax.experimental import pallas as pl
from jax.experimental.pallas import tpu as pltpu
from jax.experimental.pallas import tpu_sc as plsc

N = 8192
C_IN = 128
C_OUT = 256
K = 16
M = N // 4
EPS = 1e-5
BIGI = 2**30


_FR, _FC = 64, 128


def _fps_body(px_ref, py_ref, pz_ref, pxs_ref, pys_ref, pzs_ref,
              npx_ref, npy_ref, npz_ref):
    rows = lax.broadcasted_iota(jnp.int32, (_FR, _FC), 0)
    cols = lax.broadcasted_iota(jnp.int32, (_FR, _FC), 1)
    lin = rows * _FC + cols
    px = px_ref[...]
    py = py_ref[...]
    pz = pz_ref[...]
    qx0 = pxs_ref[0]
    qy0 = pys_ref[0]
    qz0 = pzs_ref[0]
    npx_ref[0] = qx0
    npy_ref[0] = qy0
    npz_ref[0] = qz0

    def step(i, carry):
        dists, qx, qy, qz = carry
        dx = px - qx
        dy = py - qy
        dz = pz - qz
        d = dx * dx + dy * dy + dz * dz
        dists = jnp.minimum(dists, d)
        mx = jnp.max(dists)
        nxt = jnp.min(jnp.where(dists == mx, lin, BIGI))
        nx = pxs_ref[nxt]
        ny = pys_ref[nxt]
        nz = pzs_ref[nxt]
        npx_ref[i] = nx
        npy_ref[i] = ny
        npz_ref[i] = nz
        return (dists, nx, ny, nz)

    init = (jnp.full((_FR, _FC), 1e10, jnp.float32), qx0, qy0, qz0)
    lax.fori_loop(1, M, step, init)


def _run_fps(p):
    px = p[:, 0].reshape(_FR, _FC)
    py = p[:, 1].reshape(_FR, _FC)
    pz = p[:, 2].reshape(_FR, _FC)
    vspec = pl.BlockSpec(memory_space=pltpu.MemorySpace.VMEM)
    sspec = pl.BlockSpec(memory_space=pltpu.SMEM)
    return pl.pallas_call(
        _fps_body,
        in_specs=[vspec, vspec, vspec, sspec, sspec, sspec],
        out_shape=[jax.ShapeDtypeStruct((M,), jnp.float32)] * 3,
        out_specs=[sspec] * 3,
    )(px, py, pz, p[:, 0], p[:, 1], p[:, 2])


_KT = 128
_KC = 1024
_NKC = N // _KC


def _knn_body(np_ref, pT_ref, knn_ref, d2_ref):
    cx = np_ref[:, 0:1]
    cy = np_ref[:, 1:2]
    cz = np_ref[:, 2:3]
    for c in range(_NKC):
        s = c * _KC
        dx = cx - pT_ref[0:1, s:s + _KC]
        dy = cy - pT_ref[1:2, s:s + _KC]
        dz = cz - pT_ref[2:3, s:s + _KC]
        d2_ref[:, s:s + _KC] = dx * dx + dy * dy + dz * dz
    inf = jnp.float32(jnp.inf)
    im = jnp.full((_KT, 1), -1, jnp.int32)
    for k in range(K):
        mn = jnp.full((_KT, 1), inf, jnp.float32)
        for c in range(_NKC):
            s = c * _KC
            ci = lax.broadcasted_iota(jnp.int32, (_KT, _KC), 1) + s
            blk = d2_ref[:, s:s + _KC]
            if k > 0:
                blk = jnp.where(ci == im, inf, blk)
                d2_ref[:, s:s + _KC] = blk
            mn = jnp.minimum(mn, jnp.min(blk, axis=1, keepdims=True))
        im = jnp.full((_KT, 1), BIGI, jnp.int32)
        for c in range(_NKC):
            s = c * _KC
            ci = lax.broadcasted_iota(jnp.int32, (_KT, _KC), 1) + s
            blk = d2_ref[:, s:s + _KC]
            cand = jnp.min(jnp.where(blk == mn, ci, BIGI), axis=1, keepdims=True)
            im = jnp.minimum(im, cand)
        knn_ref[:, k:k + 1] = im


def _run_knn(n_p, pT):
    return pl.pallas_call(
        _knn_body,
        grid=(M // _KT,),
        in_specs=[
            pl.BlockSpec((_KT, 3), lambda i: (i, 0)),
            pl.BlockSpec((3, N), lambda i: (0, 0)),
        ],
        out_specs=pl.BlockSpec((_KT, K), lambda i: (i, 0)),
        out_shape=jax.ShapeDtypeStruct((M, K), jnp.int32),
        scratch_shapes=[pltpu.VMEM((_KT, N), jnp.float32)],
    )(n_p, pT)


_MMB = 512
_NMM = N // _MMB


def _mm_body(x_ref, p_ref, w2_ref, wsp_ref, a_ref, h2_ref, u_ref, sums_ref,
             acc_ref):
    i = pl.program_id(0)
    xb = x_ref[...]
    h2 = jnp.dot(xb, w2_ref[...], preferred_element_type=jnp.float32)
    u = (jnp.dot(xb, wsp_ref[...], preferred_element_type=jnp.float32)
         + jnp.dot(p_ref[...], a_ref[...], preferred_element_type=jnp.float32))
    h2_ref[...] = h2
    u_ref[...] = u
    s1 = jnp.sum(h2, axis=0, keepdims=True)
    s2 = jnp.sum(h2 * h2, axis=0, keepdims=True)

    @pl.when(i == 0)
    def _():
        acc_ref[0:1, :] = s1
        acc_ref[1:2, :] = s2

    @pl.when(i > 0)
    def _():
        acc_ref[0:1, :] = acc_ref[0:1, :] + s1
        acc_ref[1:2, :] = acc_ref[1:2, :] + s2

    @pl.when(i == _NMM - 1)
    def _():
        sums_ref[...] = acc_ref[...]


def _run_mm(x, p, W2, Ws1p, A):
    return pl.pallas_call(
        _mm_body,
        grid=(_NMM,),
        in_specs=[
            pl.BlockSpec((_MMB, C_IN), lambda i: (i, 0)),
            pl.BlockSpec((_MMB, 3), lambda i: (i, 0)),
            pl.BlockSpec((C_IN, C_OUT), lambda i: (0, 0)),
            pl.BlockSpec((C_IN, C_IN), lambda i: (0, 0)),
            pl.BlockSpec((3, C_IN), lambda i: (0, 0)),
        ],
        out_specs=[
            pl.BlockSpec((_MMB, C_OUT), lambda i: (i, 0)),
            pl.BlockSpec((_MMB, C_IN), lambda i: (i, 0)),
            pl.BlockSpec((2, C_OUT), lambda i: (0, 0)),
        ],
        out_shape=[
            jax.ShapeDtypeStruct((N, C_OUT), jnp.float32),
            jax.ShapeDtypeStruct((N, C_IN), jnp.float32),
            jax.ShapeDtypeStruct((2, C_OUT), jnp.float32),
        ],
        scratch_shapes=[pltpu.VMEM((2, C_OUT), jnp.float32)],
    )(x, p, W2, Ws1p, A)


def _bnrelu_body(h_ref, sums_ref, g_ref, b_ref, y_ref, *, n_rows):
    mu = sums_ref[0:1, :] / n_rows
    var = sums_ref[1:2, :] / n_rows - mu * mu
    y = g_ref[...] * (h_ref[...] - mu) / jnp.sqrt(var + EPS) + b_ref[...]
    y_ref[...] = jnp.maximum(y, 0.0)


def _run_bnrelu(h, sums, g, b, n_rows, blk):
    rows, cols = h.shape
    return pl.pallas_call(
        functools.partial(_bnrelu_body, n_rows=n_rows),
        grid=(rows // blk,),
        in_specs=[
            pl.BlockSpec((blk, cols), lambda i: (i, 0)),
            pl.BlockSpec((2, cols), lambda i: (0, 0)),
            pl.BlockSpec((1, cols), lambda i: (0, 0)),
            pl.BlockSpec((1, cols), lambda i: (0, 0)),
        ],
        out_specs=pl.BlockSpec((blk, cols), lambda i: (i, 0)),
        out_shape=jax.ShapeDtypeStruct((rows, cols), jnp.float32),
    )(h, sums, g, b)


_NC, _NS = 2, 16
_NW = _NC * _NS
_RW = (M * K) // _NW
_GC = 256


def _sc_gather_body(v_hbm, knn_hbm, vg_hbm, idx0, idx1, idx2, idx3,
                    buf0, buf1, sem0, sem1):
    wid = lax.axis_index("s") * _NC + lax.axis_index("c")
    base = wid * _RW
    idxs = (idx0, idx1, idx2, idx3)
    for c in range(_RW // _GC):
        pltpu.sync_copy(knn_hbm.at[pl.ds(base + c * _GC, _GC)], idxs[c])
    bufs = (buf0, buf1)
    sems = (sem0, sem1)
    cps = [None, None]
    cps[0] = pltpu.async_copy(v_hbm.at[idx0], buf0, sem0)
    for c in range(_RW // _GC):
        if c + 1 < _RW // _GC:
            cps[(c + 1) % 2] = pltpu.async_copy(
                v_hbm.at[idxs[c + 1]], bufs[(c + 1) % 2], sems[(c + 1) % 2])
        cps[c % 2].wait()
        pltpu.sync_copy(bufs[c % 2], vg_hbm.at[pl.ds(base + c * _GC, _GC)])


def _run_sc_gather(v, knn):
    mesh = plsc.VectorSubcoreMesh(core_axis_name="c", subcore_axis_name="s")
    f = pl.kernel(
        _sc_gather_body,
        out_type=jax.ShapeDtypeStruct((M * K, C_IN), jnp.float32),
        mesh=mesh,
        scratch_types=[
            pltpu.VMEM((_GC,), jnp.int32),
            pltpu.VMEM((_GC,), jnp.int32),
            pltpu.VMEM((_GC,), jnp.int32),
            pltpu.VMEM((_GC,), jnp.int32),
            pltpu.VMEM((_GC, C_IN), jnp.float32),
            pltpu.VMEM((_GC, C_IN), jnp.float32),
            pltpu.SemaphoreType.DMA,
            pltpu.SemaphoreType.DMA,
        ],
    )
    return f(v, knn.reshape(M * K))


_PB = 1024
_NPB = (M * K) // _PB


def _pre_body(vg_ref, np_ref, a_ref, pre_ref, sums_ref, acc_ref):
    i = pl.program_id(0)
    w = jnp.dot(np_ref[...], a_ref[...], preferred_element_type=jnp.float32)
    r16 = lax.broadcasted_iota(jnp.int32, (_PB, _PB // K), 0) // K
    c16 = lax.broadcasted_iota(jnp.int32, (_PB, _PB // K), 1)
    e = jnp.where(r16 == c16, 1.0, 0.0)
    w_exp = jnp.dot(e, w, preferred_element_type=jnp.float32)
    pre = vg_ref[...] - w_exp
    pre_ref[...] = pre
    s1 = jnp.sum(pre, axis=0, keepdims=True)
    s2 = jnp.sum(pre * pre, axis=0, keepdims=True)

    @pl.when(i == 0)
    def _():
        acc_ref[0:1, :] = s1
        acc_ref[1:2, :] = s2

    @pl.when(i > 0)
    def _():
        acc_ref[0:1, :] = acc_ref[0:1, :] + s1
        acc_ref[1:2, :] = acc_ref[1:2, :] + s2

    @pl.when(i == _NPB - 1)
    def _():
        sums_ref[...] = acc_ref[...]


def _run_pre(vg, n_p, A):
    return pl.pallas_call(
        _pre_body,
        grid=(_NPB,),
        in_specs=[
            pl.BlockSpec((_PB, C_IN), lambda i: (i, 0)),
            pl.BlockSpec((_PB // K, 3), lambda i: (i, 0)),
            pl.BlockSpec((3, C_IN), lambda i: (0, 0)),
        ],
        out_specs=[
            pl.BlockSpec((_PB, C_IN), lambda i: (i, 0)),
            pl.BlockSpec((2, C_IN), lambda i: (0, 0)),
        ],
        out_shape=[
            jax.ShapeDtypeStruct((M * K, C_IN), jnp.float32),
            jax.ShapeDtypeStruct((2, C_IN), jnp.float32),
        ],
        scratch_shapes=[pltpu.VMEM((2, C_IN), jnp.float32)],
    )(vg, n_p, A)


_HB = 2048


def _hs_body(pre_ref, sums_ref, g_ref, b_ref, w_ref, bs_ref, s_ref):
    n_rows = jnp.float32(M * K)
    mu = sums_ref[0:1, :] / n_rows
    var = sums_ref[1:2, :] / n_rows - mu * mu
    h = g_ref[...] * (pre_ref[...] - mu) / jnp.sqrt(var + EPS) + b_ref[...]
    h = jnp.maximum(h, 0.0)
    s_ref[...] = jnp.sum(h * w_ref[...], axis=1, keepdims=True) + bs_ref[0, 0]


def _run_hs(pre, sums, gs1, bs1, Ws2, bs2):
    return pl.pallas_call(
        _hs_body,
        grid=((M * K) // _HB,),
        in_specs=[
            pl.BlockSpec((_HB, C_IN), lambda i: (i, 0)),
            pl.BlockSpec((2, C_IN), lambda i: (0, 0)),
            pl.BlockSpec((1, C_IN), lambda i: (0, 0)),
            pl.BlockSpec((1, C_IN), lambda i: (0, 0)),
            pl.BlockSpec((1, C_IN), lambda i: (0, 0)),
            pl.BlockSpec((1, 1), lambda i: (0, 0), memory_space=pltpu.SMEM),
        ],
        out_specs=pl.BlockSpec((_HB, 1), lambda i: (i, 0)),
        out_shape=jax.ShapeDtypeStruct((M * K, 1), jnp.float32),
    )(pre, sums, gs1.reshape(1, C_IN), bs1.reshape(1, C_IN),
      Ws2.reshape(1, C_IN), bs2.reshape(1, 1))


def _softmax_body(s_ref, p_ref):
    s = s_ref[...]
    mx = jnp.max(s, axis=1, keepdims=True)
    e = jnp.exp(s - mx)
    p_ref[...] = e / jnp.sum(e, axis=1, keepdims=True)


def _run_softmax(s):
    return pl.pallas_call(
        _softmax_body,
        out_shape=jax.ShapeDtypeStruct((M, K), jnp.float32),
    )(s.reshape(M, K))


_MW = M // _NW
_GM = 4


def _sc_wsum_body(y_hbm, knn_hbm, prob_hbm, out_hbm, idxv, probv, buf0, buf1,
                  outb, sem0, sem1):
    wid = lax.axis_index("s") * _NC + lax.axis_index("c")
    base = wid * _MW * K
    pltpu.sync_copy(knn_hbm.at[pl.ds(base, _MW * K)], idxv)
    pltpu.sync_copy(prob_hbm.at[pl.ds(base, _MW * K)], probv)
    bufs = (buf0, buf1)
    sems = (sem0, sem1)
    dnums = lax.GatherDimensionNumbers(
        offset_dims=(), collapsed_slice_dims=(0,), start_index_map=(0,))

    def one_m(ml, buf):
        pm = probv[pl.ds(ml * K, K)]
        accs = [jnp.zeros((16,), jnp.float32) for _ in range(C_OUT // 16)]
        for k in range(K):
            ik = jnp.zeros((K, 1), jnp.int32) + k
            pk = lax.gather(pm, ik, dnums, (1,),
                            mode=lax.GatherScatterMode.PROMISE_IN_BOUNDS)
            for r in range(C_OUT // 16):
                accs[r] = accs[r] + pk * buf[k, pl.ds(r * 16, 16)]
        for r in range(C_OUT // 16):
            outb[ml, pl.ds(r * 16, 16)] = accs[r]

    def g_body(g, _):
        cps = []
        for j in range(2):
            iv = idxv[pl.ds((g * 2 + j) * K, K)]
            cps.append(pltpu.async_copy(y_hbm.at[iv], bufs[j], sems[j]))
        for j in range(2):
            cps[j].wait()
            one_m(g * 2 + j, bufs[j])
        return 0

    lax.fori_loop(0, _MW // 2, g_body, 0)
    pltpu.sync_copy(outb, out_hbm.at[pl.ds(wid * _MW, _MW)])


def _run_sc_wsum(y, knn, prob):
    mesh = plsc.VectorSubcoreMesh(core_axis_name="c", subcore_axis_name="s")
    f = pl.kernel(
        _sc_wsum_body,
        out_type=jax.ShapeDtypeStruct((M, C_OUT), jnp.float32),
        mesh=mesh,
        scratch_types=[
            pltpu.VMEM((_MW * K,), jnp.int32),
            pltpu.VMEM((_MW * K,), jnp.float32),
            pltpu.VMEM((K, C_OUT), jnp.float32),
            pltpu.VMEM((K, C_OUT), jnp.float32),
            pltpu.VMEM((_MW, C_OUT), jnp.float32),
            pltpu.SemaphoreType.DMA,
            pltpu.SemaphoreType.DMA,
        ],
    )
    return f(y, knn.reshape(M * K), prob.reshape(M * K))


def kernel(p, x, o, W2, g2, b2, Ws1, gs1, bs1, Ws2, bs2):
    npx, npy, npz = _run_fps(p)
    n_p = jnp.stack([npx, npy, npz], axis=1)
    knn = _run_knn(n_p, p.T)
    h2, v, sums2 = _run_mm(x, p, W2, Ws1[3:], Ws1[:3])
    y = _run_bnrelu(h2, sums2, g2.reshape(1, C_OUT), b2.reshape(1, C_OUT),
                    float(N), 512)
    vg = _run_sc_gather(v, knn)
    pre, sums1 = _run_pre(vg, n_p, Ws1[:3])
    s = _run_hs(pre, sums1, gs1, bs1, Ws2, bs2)
    prob = _run_softmax(s)
    y_out = _run_sc_wsum(y, knn, prob)
    n_o = jnp.array([M], dtype=jnp.int32)
    return (n_p, y_out, n_o)


# speedup vs baseline: 13.2709x; 1.0604x over previous
import functools

import jax
import jax.numpy as jnp
from jax import lax
from jax.experimental import pallas as pl
from jax.experimental.pallas import tpu as pltpu
from jax.experimental.pallas import tpu_sc as plsc

N = 8192
C_IN = 128
C_OUT = 256
K = 16
M = N // 4
EPS = 1e-5
BIGI = 2**30


_FR, _FC = 64, 128


def _fps_body(px_ref, py_ref, pz_ref, pxs_ref, pys_ref, pzs_ref,
              npx_ref, npy_ref, npz_ref):
    rows = lax.broadcasted_iota(jnp.int32, (_FR, _FC), 0)
    cols = lax.broadcasted_iota(jnp.int32, (_FR, _FC), 1)
    lin = rows * _FC + cols
    px = px_ref[...]
    py = py_ref[...]
    pz = pz_ref[...]
    qx0 = pxs_ref[0]
    qy0 = pys_ref[0]
    qz0 = pzs_ref[0]
    npx_ref[0] = qx0
    npy_ref[0] = qy0
    npz_ref[0] = qz0

    def step(i, carry):
        dists, qx, qy, qz = carry
        dx = px - qx
        dy = py - qy
        dz = pz - qz
        d = dx * dx + dy * dy + dz * dz
        dists = jnp.minimum(dists, d)
        a = jnp.maximum(dists[0:32], dists[32:64])
        a = jnp.maximum(a[0:16], a[16:32])
        a = jnp.maximum(a[0:8], a[8:16])
        a = jnp.max(a, axis=0, keepdims=True)
        mx = jnp.max(a, axis=1, keepdims=True)
        cand = jnp.where(dists == mx, lin, BIGI)
        b = jnp.minimum(cand[0:32], cand[32:64])
        b = jnp.minimum(b[0:16], b[16:32])
        b = jnp.minimum(b[0:8], b[8:16])
        b = jnp.min(b, axis=0, keepdims=True)
        nxt = jnp.min(b, axis=1, keepdims=True)[0, 0]
        nx = pxs_ref[nxt]
        ny = pys_ref[nxt]
        nz = pzs_ref[nxt]
        npx_ref[i] = nx
        npy_ref[i] = ny
        npz_ref[i] = nz
        return (dists, nx, ny, nz)

    init = (jnp.full((_FR, _FC), 1e10, jnp.float32), qx0, qy0, qz0)
    lax.fori_loop(1, M, step, init)


def _run_fps(p):
    px = p[:, 0].reshape(_FR, _FC)
    py = p[:, 1].reshape(_FR, _FC)
    pz = p[:, 2].reshape(_FR, _FC)
    vspec = pl.BlockSpec(memory_space=pltpu.MemorySpace.VMEM)
    sspec = pl.BlockSpec(memory_space=pltpu.SMEM)
    return pl.pallas_call(
        _fps_body,
        in_specs=[vspec, vspec, vspec, sspec, sspec, sspec],
        out_shape=[jax.ShapeDtypeStruct((M,), jnp.float32)] * 3,
        out_specs=[sspec] * 3,
    )(px, py, pz, p[:, 0], p[:, 1], p[:, 2])


_KT = 128
_KC = 1024
_NKC = N // _KC


def _knn_body(np_ref, pT_ref, knn_ref, d2_ref):
    cx = np_ref[:, 0:1]
    cy = np_ref[:, 1:2]
    cz = np_ref[:, 2:3]
    for c in range(_NKC):
        s = c * _KC
        dx = cx - pT_ref[0:1, s:s + _KC]
        dy = cy - pT_ref[1:2, s:s + _KC]
        dz = cz - pT_ref[2:3, s:s + _KC]
        d2_ref[:, s:s + _KC] = dx * dx + dy * dy + dz * dz
    inf = jnp.float32(jnp.inf)
    im = jnp.full((_KT, 1), -1, jnp.int32)
    for k in range(K):
        mn = jnp.full((_KT, 1), inf, jnp.float32)
        for c in range(_NKC):
            s = c * _KC
            ci = lax.broadcasted_iota(jnp.int32, (_KT, _KC), 1) + s
            blk = d2_ref[:, s:s + _KC]
            if k > 0:
                blk = jnp.where(ci == im, inf, blk)
                d2_ref[:, s:s + _KC] = blk
            mn = jnp.minimum(mn, jnp.min(blk, axis=1, keepdims=True))
        im = jnp.full((_KT, 1), BIGI, jnp.int32)
        for c in range(_NKC):
            s = c * _KC
            ci = lax.broadcasted_iota(jnp.int32, (_KT, _KC), 1) + s
            blk = d2_ref[:, s:s + _KC]
            cand = jnp.min(jnp.where(blk == mn, ci, BIGI), axis=1, keepdims=True)
            im = jnp.minimum(im, cand)
        knn_ref[:, k:k + 1] = im


def _run_knn(n_p, pT):
    return pl.pallas_call(
        _knn_body,
        grid=(M // _KT,),
        in_specs=[
            pl.BlockSpec((_KT, 3), lambda i: (i, 0)),
            pl.BlockSpec((3, N), lambda i: (0, 0)),
        ],
        out_specs=pl.BlockSpec((_KT, K), lambda i: (i, 0)),
        out_shape=jax.ShapeDtypeStruct((M, K), jnp.int32),
        scratch_shapes=[pltpu.VMEM((_KT, N), jnp.float32)],
    )(n_p, pT)


_MMB = 512
_NMM = N // _MMB


def _mm_body(x_ref, p_ref, w2_ref, wsp_ref, a_ref, h2_ref, u_ref, sums_ref,
             acc_ref):
    i = pl.program_id(0)
    xb = x_ref[...]
    h2 = jnp.dot(xb, w2_ref[...], preferred_element_type=jnp.float32)
    u = (jnp.dot(xb, wsp_ref[...], preferred_element_type=jnp.float32)
         + jnp.dot(p_ref[...], a_ref[...], preferred_element_type=jnp.float32))
    h2_ref[...] = h2
    u_ref[...] = u
    s1 = jnp.sum(h2, axis=0, keepdims=True)
    s2 = jnp.sum(h2 * h2, axis=0, keepdims=True)

    @pl.when(i == 0)
    def _():
        acc_ref[0:1, :] = s1
        acc_ref[1:2, :] = s2

    @pl.when(i > 0)
    def _():
        acc_ref[0:1, :] = acc_ref[0:1, :] + s1
        acc_ref[1:2, :] = acc_ref[1:2, :] + s2

    @pl.when(i == _NMM - 1)
    def _():
        sums_ref[...] = acc_ref[...]


def _run_mm(x, p, W2, Ws1p, A):
    return pl.pallas_call(
        _mm_body,
        grid=(_NMM,),
        in_specs=[
            pl.BlockSpec((_MMB, C_IN), lambda i: (i, 0)),
            pl.BlockSpec((_MMB, 3), lambda i: (i, 0)),
            pl.BlockSpec((C_IN, C_OUT), lambda i: (0, 0)),
            pl.BlockSpec((C_IN, C_IN), lambda i: (0, 0)),
            pl.BlockSpec((3, C_IN), lambda i: (0, 0)),
        ],
        out_specs=[
            pl.BlockSpec((_MMB, C_OUT), lambda i: (i, 0)),
            pl.BlockSpec((_MMB, C_IN), lambda i: (i, 0)),
            pl.BlockSpec((2, C_OUT), lambda i: (0, 0)),
        ],
        out_shape=[
            jax.ShapeDtypeStruct((N, C_OUT), jnp.float32),
            jax.ShapeDtypeStruct((N, C_IN), jnp.float32),
            jax.ShapeDtypeStruct((2, C_OUT), jnp.float32),
        ],
        scratch_shapes=[pltpu.VMEM((2, C_OUT), jnp.float32)],
    )(x, p, W2, Ws1p, A)


def _bnrelu_body(h_ref, sums_ref, g_ref, b_ref, y_ref, *, n_rows):
    mu = sums_ref[0:1, :] / n_rows
    var = sums_ref[1:2, :] / n_rows - mu * mu
    y = g_ref[...] * (h_ref[...] - mu) / jnp.sqrt(var + EPS) + b_ref[...]
    y_ref[...] = jnp.maximum(y, 0.0)


def _run_bnrelu(h, sums, g, b, n_rows, blk):
    rows, cols = h.shape
    return pl.pallas_call(
        functools.partial(_bnrelu_body, n_rows=n_rows),
        grid=(rows // blk,),
        in_specs=[
            pl.BlockSpec((blk, cols), lambda i: (i, 0)),
            pl.BlockSpec((2, cols), lambda i: (0, 0)),
            pl.BlockSpec((1, cols), lambda i: (0, 0)),
            pl.BlockSpec((1, cols), lambda i: (0, 0)),
        ],
        out_specs=pl.BlockSpec((blk, cols), lambda i: (i, 0)),
        out_shape=jax.ShapeDtypeStruct((rows, cols), jnp.float32),
    )(h, sums, g, b)


_NC, _NS = 2, 16
_NW = _NC * _NS
_RW = (M * K) // _NW
_GC = 256


def _sc_gather_body(v_hbm, knn_hbm, vg_hbm, idx0, idx1, idx2, idx3,
                    buf0, buf1, sem0, sem1):
    wid = lax.axis_index("s") * _NC + lax.axis_index("c")
    base = wid * _RW
    idxs = (idx0, idx1, idx2, idx3)
    for c in range(_RW // _GC):
        pltpu.sync_copy(knn_hbm.at[pl.ds(base + c * _GC, _GC)], idxs[c])
    bufs = (buf0, buf1)
    sems = (sem0, sem1)
    cps = [None, None]
    cps[0] = pltpu.async_copy(v_hbm.at[idx0], buf0, sem0)
    for c in range(_RW // _GC):
        if c + 1 < _RW // _GC:
            cps[(c + 1) % 2] = pltpu.async_copy(
                v_hbm.at[idxs[c + 1]], bufs[(c + 1) % 2], sems[(c + 1) % 2])
        cps[c % 2].wait()
        pltpu.sync_copy(bufs[c % 2], vg_hbm.at[pl.ds(base + c * _GC, _GC)])


def _run_sc_gather(v, knn):
    mesh = plsc.VectorSubcoreMesh(core_axis_name="c", subcore_axis_name="s")
    f = pl.kernel(
        _sc_gather_body,
        out_type=jax.ShapeDtypeStruct((M * K, C_IN), jnp.float32),
        mesh=mesh,
        scratch_types=[
            pltpu.VMEM((_GC,), jnp.int32),
            pltpu.VMEM((_GC,), jnp.int32),
            pltpu.VMEM((_GC,), jnp.int32),
            pltpu.VMEM((_GC,), jnp.int32),
            pltpu.VMEM((_GC, C_IN), jnp.float32),
            pltpu.VMEM((_GC, C_IN), jnp.float32),
            pltpu.SemaphoreType.DMA,
            pltpu.SemaphoreType.DMA,
        ],
    )
    return f(v, knn.reshape(M * K))


_PB = 1024
_NPB = (M * K) // _PB


def _pre_body(vg_ref, np_ref, a_ref, pre_ref, sums_ref, acc_ref):
    i = pl.program_id(0)
    w = jnp.dot(np_ref[...], a_ref[...], preferred_element_type=jnp.float32)
    r16 = lax.broadcasted_iota(jnp.int32, (_PB, _PB // K), 0) // K
    c16 = lax.broadcasted_iota(jnp.int32, (_PB, _PB // K), 1)
    e = jnp.where(r16 == c16, 1.0, 0.0)
    w_exp = jnp.dot(e, w, preferred_element_type=jnp.float32)
    pre = vg_ref[...] - w_exp
    pre_ref[...] = pre
    s1 = jnp.sum(pre, axis=0, keepdims=True)
    s2 = jnp.sum(pre * pre, axis=0, keepdims=True)

    @pl.when(i == 0)
    def _():
        acc_ref[0:1, :] = s1
        acc_ref[1:2, :] = s2

    @pl.when(i > 0)
    def _():
        acc_ref[0:1, :] = acc_ref[0:1, :] + s1
        acc_ref[1:2, :] = acc_ref[1:2, :] + s2

    @pl.when(i == _NPB - 1)
    def _():
        sums_ref[...] = acc_ref[...]


def _run_pre(vg, n_p, A):
    return pl.pallas_call(
        _pre_body,
        grid=(_NPB,),
        in_specs=[
            pl.BlockSpec((_PB, C_IN), lambda i: (i, 0)),
            pl.BlockSpec((_PB // K, 3), lambda i: (i, 0)),
            pl.BlockSpec((3, C_IN), lambda i: (0, 0)),
        ],
        out_specs=[
            pl.BlockSpec((_PB, C_IN), lambda i: (i, 0)),
            pl.BlockSpec((2, C_IN), lambda i: (0, 0)),
        ],
        out_shape=[
            jax.ShapeDtypeStruct((M * K, C_IN), jnp.float32),
            jax.ShapeDtypeStruct((2, C_IN), jnp.float32),
        ],
        scratch_shapes=[pltpu.VMEM((2, C_IN), jnp.float32)],
    )(vg, n_p, A)


_HB = 2048


def _hs_body(pre_ref, sums_ref, g_ref, b_ref, w_ref, bs_ref, s_ref):
    n_rows = jnp.float32(M * K)
    mu = sums_ref[0:1, :] / n_rows
    var = sums_ref[1:2, :] / n_rows - mu * mu
    h = g_ref[...] * (pre_ref[...] - mu) / jnp.sqrt(var + EPS) + b_ref[...]
    h = jnp.maximum(h, 0.0)
    s_ref[...] = jnp.sum(h * w_ref[...], axis=1, keepdims=True) + bs_ref[0, 0]


def _run_hs(pre, sums, gs1, bs1, Ws2, bs2):
    return pl.pallas_call(
        _hs_body,
        grid=((M * K) // _HB,),
        in_specs=[
            pl.BlockSpec((_HB, C_IN), lambda i: (i, 0)),
            pl.BlockSpec((2, C_IN), lambda i: (0, 0)),
            pl.BlockSpec((1, C_IN), lambda i: (0, 0)),
            pl.BlockSpec((1, C_IN), lambda i: (0, 0)),
            pl.BlockSpec((1, C_IN), lambda i: (0, 0)),
            pl.BlockSpec((1, 1), lambda i: (0, 0), memory_space=pltpu.SMEM),
        ],
        out_specs=pl.BlockSpec((_HB, 1), lambda i: (i, 0)),
        out_shape=jax.ShapeDtypeStruct((M * K, 1), jnp.float32),
    )(pre, sums, gs1.reshape(1, C_IN), bs1.reshape(1, C_IN),
      Ws2.reshape(1, C_IN), bs2.reshape(1, 1))


def _softmax_body(s_ref, p_ref):
    s = s_ref[...]
    mx = jnp.max(s, axis=1, keepdims=True)
    e = jnp.exp(s - mx)
    p_ref[...] = e / jnp.sum(e, axis=1, keepdims=True)


def _run_softmax(s):
    return pl.pallas_call(
        _softmax_body,
        out_shape=jax.ShapeDtypeStruct((M, K), jnp.float32),
    )(s.reshape(M, K))


_MW = M // _NW
_GM = 4


def _sc_wsum_body(y_hbm, knn_hbm, prob_hbm, out_hbm, idxv, probv, buf0, buf1,
                  outb, sem0, sem1):
    wid = lax.axis_index("s") * _NC + lax.axis_index("c")
    base = wid * _MW * K
    pltpu.sync_copy(knn_hbm.at[pl.ds(base, _MW * K)], idxv)
    pltpu.sync_copy(prob_hbm.at[pl.ds(base, _MW * K)], probv)
    bufs = (buf0, buf1)
    sems = (sem0, sem1)
    dnums = lax.GatherDimensionNumbers(
        offset_dims=(), collapsed_slice_dims=(0,), start_index_map=(0,))

    def one_m(ml, buf):
        pm = probv[pl.ds(ml * K, K)]
        accs = [jnp.zeros((16,), jnp.float32) for _ in range(C_OUT // 16)]
        for k in range(K):
            ik = jnp.zeros((K, 1), jnp.int32) + k
            pk = lax.gather(pm, ik, dnums, (1,),
                            mode=lax.GatherScatterMode.PROMISE_IN_BOUNDS)
            for r in range(C_OUT // 16):
                accs[r] = accs[r] + pk * buf[k, pl.ds(r * 16, 16)]
        for r in range(C_OUT // 16):
            outb[ml, pl.ds(r * 16, 16)] = accs[r]

    def g_body(g, _):
        cps = []
        for j in range(2):
            iv = idxv[pl.ds((g * 2 + j) * K, K)]
            cps.append(pltpu.async_copy(y_hbm.at[iv], bufs[j], sems[j]))
        for j in range(2):
            cps[j].wait()
            one_m(g * 2 + j, bufs[j])
        return 0

    lax.fori_loop(0, _MW // 2, g_body, 0)
    pltpu.sync_copy(outb, out_hbm.at[pl.ds(wid * _MW, _MW)])


def _run_sc_wsum(y, knn, prob):
    mesh = plsc.VectorSubcoreMesh(core_axis_name="c", subcore_axis_name="s")
    f = pl.kernel(
        _sc_wsum_body,
        out_type=jax.ShapeDtypeStruct((M, C_OUT), jnp.float32),
        mesh=mesh,
        scratch_types=[
            pltpu.VMEM((_MW * K,), jnp.int32),
            pltpu.VMEM((_MW * K,), jnp.float32),
            pltpu.VMEM((K, C_OUT), jnp.float32),
            pltpu.VMEM((K, C_OUT), jnp.float32),
            pltpu.VMEM((_MW, C_OUT), jnp.float32),
            pltpu.SemaphoreType.DMA,
            pltpu.SemaphoreType.DMA,
        ],
    )
    return f(y, knn.reshape(M * K), prob.reshape(M * K))


def kernel(p, x, o, W2, g2, b2, Ws1, gs1, bs1, Ws2, bs2):
    npx, npy, npz = _run_fps(p)
    n_p = jnp.stack([npx, npy, npz], axis=1)
    knn = _run_knn(n_p, p.T)
    h2, v, sums2 = _run_mm(x, p, W2, Ws1[3:], Ws1[:3])
    y = _run_bnrelu(h2, sums2, g2.reshape(1, C_OUT), b2.reshape(1, C_OUT),
                    float(N), 512)
    vg = _run_sc_gather(v, knn)
    pre, sums1 = _run_pre(vg, n_p, Ws1[:3])
    s = _run_hs(pre, sums1, gs1, bs1, Ws2, bs2)
    prob = _run_softmax(s)
    y_out = _run_sc_wsum(y, knn, prob)
    n_o = jnp.array([M], dtype=jnp.int32)
    return (n_p, y_out, n_o)


# speedup vs baseline: 13.3213x; 1.0038x over previous
import functools

import jax
import jax.numpy as jnp
from jax import lax
from jax.experimental import pallas as pl
from jax.experimental.pallas import tpu as pltpu
from jax.experimental.pallas import tpu_sc as plsc

N = 8192
C_IN = 128
C_OUT = 256
K = 16
M = N // 4
EPS = 1e-5
BIGI = 2**30


_FR, _FC = 64, 128


def _fps_body(px_ref, py_ref, pz_ref, pxs_ref, pys_ref, pzs_ref,
              npx_ref, npy_ref, npz_ref):
    rows = lax.broadcasted_iota(jnp.int32, (_FR, _FC), 0)
    cols = lax.broadcasted_iota(jnp.int32, (_FR, _FC), 1)
    lin = rows * _FC + cols
    px = px_ref[...]
    py = py_ref[...]
    pz = pz_ref[...]
    qx0 = pxs_ref[0]
    qy0 = pys_ref[0]
    qz0 = pzs_ref[0]
    npx_ref[0] = qx0
    npy_ref[0] = qy0
    npz_ref[0] = qz0

    def step(i, carry):
        dists, qx, qy, qz = carry
        dx = px - qx
        dy = py - qy
        dz = pz - qz
        d = dx * dx + dy * dy + dz * dz
        dists = jnp.minimum(dists, d)
        a = jnp.maximum(dists[0:32], dists[32:64])
        a = jnp.maximum(a[0:16], a[16:32])
        a = jnp.maximum(a[0:8], a[8:16])
        a = jnp.max(a, axis=0, keepdims=True)
        mx = jnp.max(a, axis=1, keepdims=True)
        cand = jnp.where(dists == mx, lin, BIGI)
        b = jnp.minimum(cand[0:32], cand[32:64])
        b = jnp.minimum(b[0:16], b[16:32])
        b = jnp.minimum(b[0:8], b[8:16])
        b = jnp.min(b, axis=0, keepdims=True)
        nxt = jnp.min(b, axis=1, keepdims=True)[0, 0]
        nx = pxs_ref[nxt]
        ny = pys_ref[nxt]
        nz = pzs_ref[nxt]
        npx_ref[i] = nx
        npy_ref[i] = ny
        npz_ref[i] = nz
        return (dists, nx, ny, nz)

    init = (jnp.full((_FR, _FC), 1e10, jnp.float32), qx0, qy0, qz0)
    lax.fori_loop(1, M, step, init)


def _run_fps(p):
    px = p[:, 0].reshape(_FR, _FC)
    py = p[:, 1].reshape(_FR, _FC)
    pz = p[:, 2].reshape(_FR, _FC)
    vspec = pl.BlockSpec(memory_space=pltpu.MemorySpace.VMEM)
    sspec = pl.BlockSpec(memory_space=pltpu.SMEM)
    return pl.pallas_call(
        _fps_body,
        in_specs=[vspec, vspec, vspec, sspec, sspec, sspec],
        out_shape=[jax.ShapeDtypeStruct((M,), jnp.float32)] * 3,
        out_specs=[sspec] * 3,
    )(px, py, pz, p[:, 0], p[:, 1], p[:, 2])


_KT = 128
_KC = 1024
_NKC = N // _KC


def _knn_body(np_ref, pT_ref, knn_ref, d2_ref):
    cx = np_ref[:, 0:1]
    cy = np_ref[:, 1:2]
    cz = np_ref[:, 2:3]
    for c in range(_NKC):
        s = c * _KC
        dx = cx - pT_ref[0:1, s:s + _KC]
        dy = cy - pT_ref[1:2, s:s + _KC]
        dz = cz - pT_ref[2:3, s:s + _KC]
        d2_ref[:, s:s + _KC] = dx * dx + dy * dy + dz * dz
    inf = jnp.float32(jnp.inf)
    im = jnp.full((_KT, 1), -1, jnp.int32)
    for k in range(K):
        mn = jnp.full((_KT, 1), inf, jnp.float32)
        for c in range(_NKC):
            s = c * _KC
            ci = lax.broadcasted_iota(jnp.int32, (_KT, _KC), 1) + s
            blk = d2_ref[:, s:s + _KC]
            if k > 0:
                blk = jnp.where(ci == im, inf, blk)
                d2_ref[:, s:s + _KC] = blk
            mn = jnp.minimum(mn, jnp.min(blk, axis=1, keepdims=True))
        im = jnp.full((_KT, 1), BIGI, jnp.int32)
        for c in range(_NKC):
            s = c * _KC
            ci = lax.broadcasted_iota(jnp.int32, (_KT, _KC), 1) + s
            blk = d2_ref[:, s:s + _KC]
            cand = jnp.min(jnp.where(blk == mn, ci, BIGI), axis=1, keepdims=True)
            im = jnp.minimum(im, cand)
        knn_ref[:, k:k + 1] = im


def _run_knn(n_p, pT):
    return pl.pallas_call(
        _knn_body,
        grid=(M // _KT,),
        in_specs=[
            pl.BlockSpec((_KT, 3), lambda i: (i, 0)),
            pl.BlockSpec((3, N), lambda i: (0, 0)),
        ],
        out_specs=pl.BlockSpec((_KT, K), lambda i: (i, 0)),
        out_shape=jax.ShapeDtypeStruct((M, K), jnp.int32),
        scratch_shapes=[pltpu.VMEM((_KT, N), jnp.float32)],
    )(n_p, pT)


_MMB = 512
_NMM = N // _MMB


def _mm_body(x_ref, p_ref, w2_ref, wsp_ref, a_ref, g_ref, b_ref,
             y_ref, u_ref, h2_ref, acc_ref):
    ph = pl.program_id(0)
    i = pl.program_id(1)

    @pl.when(ph == 0)
    def _():
        xb = x_ref[...]
        h2 = jnp.dot(xb, w2_ref[...], preferred_element_type=jnp.float32)
        u = (jnp.dot(xb, wsp_ref[...], preferred_element_type=jnp.float32)
             + jnp.dot(p_ref[...], a_ref[...],
                       preferred_element_type=jnp.float32))
        h2_ref[pl.ds(i * _MMB, _MMB), :] = h2
        u_ref[...] = u
        s1 = jnp.sum(h2, axis=0, keepdims=True)
        s2 = jnp.sum(h2 * h2, axis=0, keepdims=True)

        @pl.when(i == 0)
        def _():
            acc_ref[0:1, :] = s1
            acc_ref[1:2, :] = s2

        @pl.when(i > 0)
        def _():
            acc_ref[0:1, :] = acc_ref[0:1, :] + s1
            acc_ref[1:2, :] = acc_ref[1:2, :] + s2

    @pl.when(ph == 1)
    def _():
        mu = acc_ref[0:1, :] / float(N)
        var = acc_ref[1:2, :] / float(N) - mu * mu
        y = (g_ref[...] * (h2_ref[pl.ds(i * _MMB, _MMB), :] - mu) / jnp.sqrt(var + EPS)
             + b_ref[...])
        y_ref[...] = jnp.maximum(y, 0.0)


def _run_mm(x, p, W2, Ws1p, A, g2, b2):
    return pl.pallas_call(
        _mm_body,
        grid=(2, _NMM),
        in_specs=[
            pl.BlockSpec((_MMB, C_IN), lambda p_, i: (i, 0)),
            pl.BlockSpec((_MMB, 3), lambda p_, i: (i, 0)),
            pl.BlockSpec((C_IN, C_OUT), lambda p_, i: (0, 0)),
            pl.BlockSpec((C_IN, C_IN), lambda p_, i: (0, 0)),
            pl.BlockSpec((3, C_IN), lambda p_, i: (0, 0)),
            pl.BlockSpec((1, C_OUT), lambda p_, i: (0, 0)),
            pl.BlockSpec((1, C_OUT), lambda p_, i: (0, 0)),
        ],
        out_specs=[
            pl.BlockSpec((_MMB, C_OUT),
                         lambda p_, i: (jnp.where(p_ == 1, i, 0), 0)),
            pl.BlockSpec((_MMB, C_IN),
                         lambda p_, i: (jnp.where(p_ == 0, i, _NMM - 1), 0)),
        ],
        out_shape=[
            jax.ShapeDtypeStruct((N, C_OUT), jnp.float32),
            jax.ShapeDtypeStruct((N, C_IN), jnp.float32),
        ],
        scratch_shapes=[
            pltpu.VMEM((N, C_OUT), jnp.float32),
            pltpu.VMEM((2, C_OUT), jnp.float32),
        ],
    )(x, p, W2, Ws1p, A, g2.reshape(1, C_OUT), b2.reshape(1, C_OUT))


_NC, _NS = 2, 16
_NW = _NC * _NS
_RW = (M * K) // _NW
_GC = 256


def _sc_gather_body(v_hbm, knn_hbm, vg_hbm, idx0, idx1, idx2, idx3,
                    buf0, buf1, sem0, sem1):
    wid = lax.axis_index("s") * _NC + lax.axis_index("c")
    base = wid * _RW
    idxs = (idx0, idx1, idx2, idx3)
    for c in range(_RW // _GC):
        pltpu.sync_copy(knn_hbm.at[pl.ds(base + c * _GC, _GC)], idxs[c])
    bufs = (buf0, buf1)
    sems = (sem0, sem1)
    cps = [None, None]
    cps[0] = pltpu.async_copy(v_hbm.at[idx0], buf0, sem0)
    for c in range(_RW // _GC):
        if c + 1 < _RW // _GC:
            cps[(c + 1) % 2] = pltpu.async_copy(
                v_hbm.at[idxs[c + 1]], bufs[(c + 1) % 2], sems[(c + 1) % 2])
        cps[c % 2].wait()
        pltpu.sync_copy(bufs[c % 2], vg_hbm.at[pl.ds(base + c * _GC, _GC)])


def _run_sc_gather(v, knn):
    mesh = plsc.VectorSubcoreMesh(core_axis_name="c", subcore_axis_name="s")
    f = pl.kernel(
        _sc_gather_body,
        out_type=jax.ShapeDtypeStruct((M * K, C_IN), jnp.float32),
        mesh=mesh,
        scratch_types=[
            pltpu.VMEM((_GC,), jnp.int32),
            pltpu.VMEM((_GC,), jnp.int32),
            pltpu.VMEM((_GC,), jnp.int32),
            pltpu.VMEM((_GC,), jnp.int32),
            pltpu.VMEM((_GC, C_IN), jnp.float32),
            pltpu.VMEM((_GC, C_IN), jnp.float32),
            pltpu.SemaphoreType.DMA,
            pltpu.SemaphoreType.DMA,
        ],
    )
    return f(v, knn.reshape(M * K))


_PB = 2048
_NPB = (M * K) // _PB


def _shr_body(vg_ref, np_ref, a_ref, g_ref, b_ref, w_ref, bs_ref,
              prob_ref, pre_ref, acc_ref):
    ph = pl.program_id(0)
    i = pl.program_id(1)
    nm = _PB // K
    r16 = lax.broadcasted_iota(jnp.int32, (_PB, nm), 0) // K
    c16 = lax.broadcasted_iota(jnp.int32, (_PB, nm), 1)
    e01 = jnp.where(r16 == c16, 1.0, 0.0)

    @pl.when(ph == 0)
    def _():
        w = jnp.dot(np_ref[...], a_ref[...],
                    preferred_element_type=jnp.float32)
        w_exp = jnp.dot(e01, w, preferred_element_type=jnp.float32)
        pre = vg_ref[...] - w_exp
        pre_ref[pl.ds(i * _PB, _PB), :] = pre
        s1 = jnp.sum(pre, axis=0, keepdims=True)
        s2 = jnp.sum(pre * pre, axis=0, keepdims=True)

        @pl.when(i == 0)
        def _():
            acc_ref[0:1, :] = s1
            acc_ref[1:2, :] = s2

        @pl.when(i > 0)
        def _():
            acc_ref[0:1, :] = acc_ref[0:1, :] + s1
            acc_ref[1:2, :] = acc_ref[1:2, :] + s2

    @pl.when(ph == 1)
    def _():
        n_rows = float(M * K)
        mu = acc_ref[0:1, :] / n_rows
        var = acc_ref[1:2, :] / n_rows - mu * mu
        h = (g_ref[...] * (pre_ref[pl.ds(i * _PB, _PB), :] - mu)
             / jnp.sqrt(var + EPS) + b_ref[...])
        h = jnp.maximum(h, 0.0)
        s = jnp.sum(h * w_ref[...], axis=1, keepdims=True) + bs_ref[0, 0]
        ex = jnp.exp(s)
        seg = jnp.dot(e01.T, ex, preferred_element_type=jnp.float32)
        den = jnp.dot(e01, seg, preferred_element_type=jnp.float32)
        prob_ref[...] = ex / den


def _run_shr(vg, n_p, A, gs1, bs1, Ws2, bs2):
    return pl.pallas_call(
        _shr_body,
        grid=(2, _NPB),
        in_specs=[
            pl.BlockSpec((_PB, C_IN), lambda p_, i: (i, 0)),
            pl.BlockSpec((_PB // K, 3), lambda p_, i: (i, 0)),
            pl.BlockSpec((3, C_IN), lambda p_, i: (0, 0)),
            pl.BlockSpec((1, C_IN), lambda p_, i: (0, 0)),
            pl.BlockSpec((1, C_IN), lambda p_, i: (0, 0)),
            pl.BlockSpec((1, C_IN), lambda p_, i: (0, 0)),
            pl.BlockSpec((1, 1), lambda p_, i: (0, 0),
                         memory_space=pltpu.SMEM),
        ],
        out_specs=pl.BlockSpec(
            (_PB, 1), lambda p_, i: (jnp.where(p_ == 1, i, 0), 0)),
        out_shape=jax.ShapeDtypeStruct((M * K, 1), jnp.float32),
        scratch_shapes=[
            pltpu.VMEM((M * K, C_IN), jnp.float32),
            pltpu.VMEM((2, C_IN), jnp.float32),
        ],
    )(vg, n_p, A, gs1.reshape(1, C_IN), bs1.reshape(1, C_IN),
      Ws2.reshape(1, C_IN), bs2.reshape(1, 1))


_MW = M // _NW
_GM = 4


def _sc_wsum_body(y_hbm, knn_hbm, prob_hbm, out_hbm, idxv, probv, buf0, buf1,
                  outb, sem0, sem1):
    wid = lax.axis_index("s") * _NC + lax.axis_index("c")
    base = wid * _MW * K
    pltpu.sync_copy(knn_hbm.at[pl.ds(base, _MW * K)], idxv)
    pltpu.sync_copy(prob_hbm.at[pl.ds(base, _MW * K)], probv)
    bufs = (buf0, buf1)
    sems = (sem0, sem1)
    dnums = lax.GatherDimensionNumbers(
        offset_dims=(), collapsed_slice_dims=(0,), start_index_map=(0,))

    def one_m(ml, buf):
        pm = probv[pl.ds(ml * K, K)]
        accs = [jnp.zeros((16,), jnp.float32) for _ in range(C_OUT // 16)]
        for k in range(K):
            ik = jnp.zeros((K, 1), jnp.int32) + k
            pk = lax.gather(pm, ik, dnums, (1,),
                            mode=lax.GatherScatterMode.PROMISE_IN_BOUNDS)
            for r in range(C_OUT // 16):
                accs[r] = accs[r] + pk * buf[k, pl.ds(r * 16, 16)]
        for r in range(C_OUT // 16):
            outb[ml, pl.ds(r * 16, 16)] = accs[r]

    def g_body(g, _):
        cps = []
        for j in range(2):
            iv = idxv[pl.ds((g * 2 + j) * K, K)]
            cps.append(pltpu.async_copy(y_hbm.at[iv], bufs[j], sems[j]))
        for j in range(2):
            cps[j].wait()
            one_m(g * 2 + j, bufs[j])
        return 0

    lax.fori_loop(0, _MW // 2, g_body, 0)
    pltpu.sync_copy(outb, out_hbm.at[pl.ds(wid * _MW, _MW)])


def _run_sc_wsum(y, knn, prob):
    mesh = plsc.VectorSubcoreMesh(core_axis_name="c", subcore_axis_name="s")
    f = pl.kernel(
        _sc_wsum_body,
        out_type=jax.ShapeDtypeStruct((M, C_OUT), jnp.float32),
        mesh=mesh,
        scratch_types=[
            pltpu.VMEM((_MW * K,), jnp.int32),
            pltpu.VMEM((_MW * K,), jnp.float32),
            pltpu.VMEM((K, C_OUT), jnp.float32),
            pltpu.VMEM((K, C_OUT), jnp.float32),
            pltpu.VMEM((_MW, C_OUT), jnp.float32),
            pltpu.SemaphoreType.DMA,
            pltpu.SemaphoreType.DMA,
        ],
    )
    return f(y, knn.reshape(M * K), prob.reshape(M * K))


def kernel(p, x, o, W2, g2, b2, Ws1, gs1, bs1, Ws2, bs2):
    npx, npy, npz = _run_fps(p)
    n_p = jnp.stack([npx, npy, npz], axis=1)
    knn = _run_knn(n_p, p.T)
    y, v = _run_mm(x, p, W2, Ws1[3:], Ws1[:3], g2, b2)
    vg = _run_sc_gather(v, knn)
    prob = _run_shr(vg, n_p, Ws1[:3], gs1, bs1, Ws2, bs2)
    y_out = _run_sc_wsum(y, knn, prob)
    n_o = jnp.array([M], dtype=jnp.int32)
    return (n_p, y_out, n_o)
